# baseline (speedup 1.0000x reference)
import functools

import jax
import jax.numpy as jnp
from jax import lax
from jax.experimental import pallas as pl
from jax.experimental.pallas import tpu as pltpu

_B = 4096
_D = 512
_N = 1000
_R = 512
_G = _B // _R


def _tc_body(labels_ref, feats_ref, centers_t_ref, out_ref, ct_bf, cn_row):
    i = pl.program_id(0)

    @pl.when(i == 0)
    def _():
        Ct = centers_t_ref[...]
        ct_bf[...] = (-2.0 * Ct).astype(jnp.bfloat16)
        cn_row[...] = jnp.sum(Ct * Ct, axis=0, keepdims=True)

    F = feats_ref[...]
    f2 = jnp.sum(F * F)
    P = lax.dot_general(
        F.astype(jnp.bfloat16), ct_bf[...],
        (((1,), (0,)), ((), ())), preferred_element_type=jnp.float32)
    lab = labels_ref[...]
    col = lax.broadcasted_iota(jnp.int32, (_R, _N), 1)
    mask = col == lab
    contrib = jnp.reshape(
        jnp.sum(jnp.where(mask, cn_row[...] + P, 0.0)) + f2, (1, 1))

    @pl.when(i == 0)
    def _():
        out_ref[...] = contrib

    @pl.when(i > 0)
    def _():
        out_ref[...] += contrib


def kernel(feats, labels, centers):
    lab2 = labels.astype(jnp.int32).reshape(_B, 1)
    out = pl.pallas_call(
        _tc_body,
        grid=(_G,),
        in_specs=[
            pl.BlockSpec((_R, 1), lambda i: (i, 0)),
            pl.BlockSpec((_R, _D), lambda i: (i, 0)),
            pl.BlockSpec((_D, _N), lambda i: (0, 0)),
        ],
        out_specs=pl.BlockSpec((1, 1), lambda i: (0, 0)),
        out_shape=jax.ShapeDtypeStruct((1, 1), jnp.float32),
        scratch_shapes=[
            pltpu.VMEM((_D, _N), jnp.bfloat16),
            pltpu.VMEM((1, _N), jnp.float32),
        ],
    )(lab2, feats, centers.T)
    return out[0, 0] / jnp.float32(_B * _D)
